# baseline (device time: 29007 ns/iter reference)
import jax
import jax.numpy as jnp
from jax import lax
from jax.experimental import pallas as pl
from jax.experimental.pallas import tpu as pltpu

N_DEV = 8
B = 2
SQ = 256
SKV = 256
H_PER = 4
DH = 64
D_MODEL = 512


def kernel(x, Wq, K_ext, V_ext, Wo):
    K2 = K_ext.reshape(B, SKV, 32 * DH)
    V2 = V_ext.reshape(B, SKV, 32 * DH)

    ROWS = SQ // N_DEV
    SEND_ORDER = [4, 3, 5, 2, 6, 1, 7]

    def body(x_ref, wq_ref, k_ref, v_ref, wo_ref, out_ref,
             p_buf, kv_buf, rs_recv, ssems, rsems1, rsems2, kv_sems):
        my = lax.axis_index("i")

        bar = pltpu.get_barrier_semaphore()
        for o in range(1, N_DEV):
            pl.semaphore_signal(bar, inc=1, device_id=((my + o) % N_DEV,),
                                device_id_type=pl.DeviceIdType.MESH)

        kv_start = my * (H_PER * DH)
        ck = pltpu.make_async_copy(
            k_ref.at[:, :, pl.ds(kv_start, H_PER * DH)], kv_buf.at[0],
            kv_sems.at[0])
        cv = pltpu.make_async_copy(
            v_ref.at[:, :, pl.ds(kv_start, H_PER * DH)], kv_buf.at[1],
            kv_sems.at[1])
        ck.start()
        cv.start()

        qb = lax.broadcasted_iota(jnp.int32, (SQ, SKV), 0) // 64
        kb = lax.broadcasted_iota(jnp.int32, (SQ, SKV), 1) // 64
        mask = (qb == kb) | (kb == 0) | ((qb + kb) % 3 == 0)

        q_all = [jnp.dot(x_ref[b, :, :], wq_ref[:, :],
                         preferred_element_type=jnp.float32) * 0.125
                 for b in range(B)]
        ck.wait()
        cv.wait()

        for b in range(B):
            ctx_heads = []
            for h in range(H_PER):
                qh = q_all[b][:, h * DH:(h + 1) * DH]
                kh = kv_buf[0, b, :, h * DH:(h + 1) * DH]
                s = lax.dot_general(
                    qh, kh, (((1,), (1,)), ((), ())),
                    preferred_element_type=jnp.float32)
                w = jnp.exp(jnp.where(mask, s, -1e9))
                ctx_h = jnp.dot(w, kv_buf[1, b, :, h * DH:(h + 1) * DH],
                                preferred_element_type=jnp.float32)
                ctx_heads.append(ctx_h / jnp.sum(w, axis=1, keepdims=True))
            ctx_b = jnp.concatenate(ctx_heads, axis=1)
            p_buf[b, :, :] = jnp.dot(ctx_b, wo_ref[:, :],
                                     preferred_element_type=jnp.float32)

        pl.semaphore_wait(bar, N_DEV - 1)

        rdmas1 = {}
        for o in SEND_ORDER:
            tgt = (my + o) % N_DEV
            r = pltpu.make_async_remote_copy(
                src_ref=p_buf.at[:, pl.ds(tgt * ROWS, ROWS), :],
                dst_ref=rs_recv.at[o],
                send_sem=ssems.at[o],
                recv_sem=rsems1.at[o],
                device_id=(tgt,),
                device_id_type=pl.DeviceIdType.MESH,
            )
            r.start()
            rdmas1[o] = r

        acc = p_buf[:, pl.ds(my * ROWS, ROWS), :]
        for o in range(1, N_DEV):
            rdmas1[o].wait_recv()
            acc = acc + rs_recv[o, :, :, :]
        out_ref[:, pl.ds(my * ROWS, ROWS), :] = acc
        for r in rdmas1.values():
            r.wait_send()

        rdmas2 = []
        for o in SEND_ORDER:
            tgt = (my + o) % N_DEV
            r = pltpu.make_async_remote_copy(
                src_ref=out_ref.at[:, pl.ds(my * ROWS, ROWS), :],
                dst_ref=out_ref.at[:, pl.ds(my * ROWS, ROWS), :],
                send_sem=ssems.at[o],
                recv_sem=rsems2.at[o],
                device_id=(tgt,),
                device_id_type=pl.DeviceIdType.MESH,
            )
            r.start()
            rdmas2.append(r)
        for r in rdmas2:
            r.wait_recv()
        for r in rdmas2:
            r.wait_send()

    return pl.pallas_call(
        body,
        out_shape=jax.ShapeDtypeStruct((B, SQ, D_MODEL), jnp.float32),
        in_specs=[
            pl.BlockSpec(memory_space=pltpu.VMEM),
            pl.BlockSpec(memory_space=pltpu.VMEM),
            pl.BlockSpec(memory_space=pltpu.MemorySpace.HBM),
            pl.BlockSpec(memory_space=pltpu.MemorySpace.HBM),
            pl.BlockSpec(memory_space=pltpu.VMEM),
        ],
        out_specs=pl.BlockSpec(memory_space=pltpu.VMEM),
        scratch_shapes=[
            pltpu.VMEM((B, SQ, D_MODEL), jnp.float32),
            pltpu.VMEM((2, B, SKV, H_PER * DH), jnp.float32),
            pltpu.VMEM((N_DEV, B, ROWS, D_MODEL), jnp.float32),
            pltpu.SemaphoreType.DMA((N_DEV,)),
            pltpu.SemaphoreType.DMA((N_DEV,)),
            pltpu.SemaphoreType.DMA((N_DEV,)),
            pltpu.SemaphoreType.DMA((2,)),
        ],
        compiler_params=pltpu.CompilerParams(collective_id=0),
    )(x, Wq, K2, V2, Wo)


# device time: 23738 ns/iter; 1.2220x vs baseline; 1.2220x over previous
import jax
import jax.numpy as jnp
from jax import lax
from jax.experimental import pallas as pl
from jax.experimental.pallas import tpu as pltpu

N_DEV = 8
B = 2
SQ = 256
SKV = 256
H_PER = 4
DH = 64
D_MODEL = 512


def kernel(x, Wq, K_ext, V_ext, Wo):
    idx = lax.axis_index("i")
    K_loc = lax.dynamic_slice_in_dim(
        K_ext.reshape(B, SKV, 32 * DH), idx * (H_PER * DH), H_PER * DH, axis=2)
    V_loc = lax.dynamic_slice_in_dim(
        V_ext.reshape(B, SKV, 32 * DH), idx * (H_PER * DH), H_PER * DH, axis=2)

    ROWS = SQ // N_DEV
    SEND_ORDER = [4, 3, 5, 2, 6, 1, 7]

    def body(x_ref, wq_ref, k_ref, v_ref, wo_ref, out_ref,
             p_buf, rs_recv, ssems, rsems1, rsems2):
        my = lax.axis_index("i")

        bar = pltpu.get_barrier_semaphore()
        for o in range(1, N_DEV):
            pl.semaphore_signal(bar, inc=1, device_id=((my + o) % N_DEV,),
                                device_id_type=pl.DeviceIdType.MESH)

        qb = lax.broadcasted_iota(jnp.int32, (SQ, SKV), 0) // 64
        kb = lax.broadcasted_iota(jnp.int32, (SQ, SKV), 1) // 64
        mask = (qb == kb) | (kb == 0) | ((qb + kb) % 3 == 0)

        for b in range(B):
            q_b = jnp.dot(x_ref[b, :, :], wq_ref[:, :],
                          preferred_element_type=jnp.float32) * 0.125
            ctx_heads = []
            for h in range(H_PER):
                qh = q_b[:, h * DH:(h + 1) * DH]
                kh = k_ref[b, :, h * DH:(h + 1) * DH]
                s = lax.dot_general(
                    qh, kh, (((1,), (1,)), ((), ())),
                    preferred_element_type=jnp.float32)
                w = jnp.exp(jnp.where(mask, s, -1e9))
                ctx_h = jnp.dot(w, v_ref[b, :, h * DH:(h + 1) * DH],
                                preferred_element_type=jnp.float32)
                ctx_heads.append(ctx_h / jnp.sum(w, axis=1, keepdims=True))
            ctx_b = jnp.concatenate(ctx_heads, axis=1)
            p_buf[b, :, :] = jnp.dot(ctx_b, wo_ref[:, :],
                                     preferred_element_type=jnp.float32)

        pl.semaphore_wait(bar, N_DEV - 1)

        rdmas1 = {}
        for o in SEND_ORDER:
            tgt = (my + o) % N_DEV
            r = pltpu.make_async_remote_copy(
                src_ref=p_buf.at[:, pl.ds(tgt * ROWS, ROWS), :],
                dst_ref=rs_recv.at[o],
                send_sem=ssems.at[o],
                recv_sem=rsems1.at[o],
                device_id=(tgt,),
                device_id_type=pl.DeviceIdType.MESH,
            )
            r.start()
            rdmas1[o] = r

        acc = p_buf[:, pl.ds(my * ROWS, ROWS), :]
        for o in range(1, N_DEV):
            rdmas1[o].wait_recv()
            acc = acc + rs_recv[o, :, :, :]
        out_ref[:, pl.ds(my * ROWS, ROWS), :] = acc
        for r in rdmas1.values():
            r.wait_send()

        rdmas2 = []
        for o in SEND_ORDER:
            tgt = (my + o) % N_DEV
            r = pltpu.make_async_remote_copy(
                src_ref=out_ref.at[:, pl.ds(my * ROWS, ROWS), :],
                dst_ref=out_ref.at[:, pl.ds(my * ROWS, ROWS), :],
                send_sem=ssems.at[o],
                recv_sem=rsems2.at[o],
                device_id=(tgt,),
                device_id_type=pl.DeviceIdType.MESH,
            )
            r.start()
            rdmas2.append(r)
        for r in rdmas2:
            r.wait_recv()
        for r in rdmas2:
            r.wait_send()

    return pl.pallas_call(
        body,
        out_shape=jax.ShapeDtypeStruct((B, SQ, D_MODEL), jnp.float32),
        in_specs=[pl.BlockSpec(memory_space=pltpu.VMEM)] * 5,
        out_specs=pl.BlockSpec(memory_space=pltpu.VMEM),
        scratch_shapes=[
            pltpu.VMEM((B, SQ, D_MODEL), jnp.float32),
            pltpu.VMEM((N_DEV, B, ROWS, D_MODEL), jnp.float32),
            pltpu.SemaphoreType.DMA((N_DEV,)),
            pltpu.SemaphoreType.DMA((N_DEV,)),
            pltpu.SemaphoreType.DMA((N_DEV,)),
        ],
        compiler_params=pltpu.CompilerParams(collective_id=0),
    )(x, Wq, K_loc, V_loc, Wo)
